# baseline (speedup 1.0000x reference)
import functools
import math

import jax
import jax.numpy as jnp
from jax.experimental import pallas as pl
from jax.experimental.pallas import tpu as pltpu
from jax.experimental.pallas import tpu_sc as plsc

_SIZE = 100000
_CONF = 0.9
_EPS = float(jnp.float32(0.1 / (_SIZE - 2)))
_H = _CONF * math.log(_CONF) + (_SIZE - 2) * _EPS * math.log(_EPS)
_ROWS_PER_BLOCK = 8


def _sc_gather(xflat, idx):
    info = plsc.get_sparse_core_info()
    nc, ns = info.num_cores, info.num_subcores
    nw = nc * ns
    n = idx.shape[0]
    bpw = n // nw
    mesh = plsc.VectorSubcoreMesh(core_axis_name="c", subcore_axis_name="s")

    @functools.partial(
        pl.kernel,
        out_type=jax.ShapeDtypeStruct((n,), jnp.float32),
        mesh=mesh,
        scratch_types=[
            pltpu.VMEM((bpw,), jnp.int32),
            pltpu.VMEM((bpw,), jnp.float32),
            pltpu.SemaphoreType.DMA,
        ],
    )
    def gather_k(xflat_hbm, idx_hbm, out_hbm, idx_v, vals_v, sem):
        wid = jax.lax.axis_index("s") * nc + jax.lax.axis_index("c")
        base = wid * bpw
        pltpu.sync_copy(idx_hbm.at[pl.ds(base, bpw)], idx_v)
        pltpu.async_copy(xflat_hbm.at[idx_v], vals_v, sem).wait()
        pltpu.sync_copy(vals_v, out_hbm.at[pl.ds(base, bpw)])

    return gather_k(xflat, idx)


def _tc_body(t_ref, v_ref, x_ref, o_ref):
    pid = pl.program_id(0)
    t = t_ref[...]
    w = (t != 0).astype(jnp.float32)
    v = v_ref[...]
    x = x_ref[...]
    rowsum = jnp.sum(x, axis=1, keepdims=True)
    contrib = jnp.sum(
        w * (_H - (_CONF - _EPS) * v - _EPS * (rowsum - x[:, 0:1]))
    )

    @pl.when(pid == 0)
    def _init():
        o_ref[0, 0] = 0.0

    o_ref[0, 0] += contrib


def kernel(x, target):
    n = x.shape[0]
    r = _ROWS_PER_BLOCK
    t32 = target.astype(jnp.int32)
    idx = t32 + jnp.arange(n, dtype=jnp.int32) * _SIZE
    vals = _sc_gather(x.reshape(-1), idx)
    out = pl.pallas_call(
        _tc_body,
        grid=(n // r,),
        in_specs=[
            pl.BlockSpec((r, 1), lambda i: (i, 0)),
            pl.BlockSpec((r, 1), lambda i: (i, 0)),
            pl.BlockSpec((r, _SIZE), lambda i: (i, 0)),
        ],
        out_specs=pl.BlockSpec(memory_space=pltpu.SMEM),
        out_shape=jax.ShapeDtypeStruct((1, 1), jnp.float32),
    )(t32.reshape(n, 1), vals.reshape(n, 1), x)
    return out[0, 0]


# speedup vs baseline: 2.0961x; 2.0961x over previous
import math

import jax
import jax.numpy as jnp
from jax.experimental import pallas as pl
from jax.experimental.pallas import tpu as pltpu

_SIZE = 100000
_CONF = 0.9
_EPS = float(jnp.float32(0.1 / (_SIZE - 2)))
_H = _CONF * math.log(_CONF) + (_SIZE - 2) * _EPS * math.log(_EPS)
_ROWS_PER_BLOCK = 8


def _tc_body(t_ref, x_ref, o_ref):
    pid = pl.program_id(0)
    x = x_ref[...]
    rowsum = jnp.sum(x, axis=1)
    lane_iota = jax.lax.broadcasted_iota(jnp.int32, (1, 128), 1)
    contrib = 0.0
    for k in range(_ROWS_PER_BLOCK):
        tk = t_ref[pid * _ROWS_PER_BLOCK + k]
        wk = (tk != 0).astype(jnp.float32)
        col0 = pl.multiple_of((tk // 128) * 128, 128)
        window = x_ref[pl.ds(k, 1), pl.ds(col0, 128)]
        lane = tk - col0
        vk = jnp.sum(jnp.where(lane_iota == lane, window, 0.0))
        x0k = x_ref[k, 0]
        contrib += wk * (
            _H - (_CONF - _EPS) * vk - _EPS * (rowsum[k] - x0k)
        )

    @pl.when(pid == 0)
    def _init():
        o_ref[0, 0] = 0.0

    o_ref[0, 0] += contrib


def kernel(x, target):
    n = x.shape[0]
    r = _ROWS_PER_BLOCK
    t32 = target.astype(jnp.int32)
    out = pl.pallas_call(
        _tc_body,
        grid=(n // r,),
        in_specs=[
            pl.BlockSpec((n,), lambda i: (0,), memory_space=pltpu.SMEM),
            pl.BlockSpec((r, _SIZE), lambda i: (i, 0)),
        ],
        out_specs=pl.BlockSpec(memory_space=pltpu.SMEM),
        out_shape=jax.ShapeDtypeStruct((1, 1), jnp.float32),
    )(t32, x)
    return out[0, 0]


# speedup vs baseline: 2.3645x; 1.1281x over previous
import math

import jax
import jax.numpy as jnp
import numpy as np
from jax.experimental import pallas as pl
from jax.experimental.pallas import tpu as pltpu

_SIZE = 100000
_CONF = 0.9
_EPS = float(np.float32(0.1 / (_SIZE - 2)))
_H = _CONF * math.log(_CONF) + (_SIZE - 2) * _EPS * math.log(_EPS)
_ROWS_PER_BLOCK = 16


def _tc_body(t_ref, x_ref, o_ref):
    pid = pl.program_id(0)
    x = x_ref[...]
    rowsum = jnp.sum(x, axis=1)
    lane_iota = jax.lax.broadcasted_iota(jnp.int32, (1, 128), 1)
    contrib = 0.0
    for k in range(_ROWS_PER_BLOCK):
        tk = t_ref[pid * _ROWS_PER_BLOCK + k]
        wk = (tk != 0).astype(jnp.float32)
        col0 = pl.multiple_of((tk // 128) * 128, 128)
        window = x_ref[pl.ds(k, 1), pl.ds(col0, 128)]
        lane = tk - col0
        vk = jnp.sum(jnp.where(lane_iota == lane, window, 0.0))
        x0k = x_ref[k, 0]
        contrib += wk * (
            _H - (_CONF - _EPS) * vk - _EPS * (rowsum[k] - x0k)
        )

    @pl.when(pid == 0)
    def _init():
        o_ref[0, 0] = 0.0

    o_ref[0, 0] += contrib


def kernel(x, target):
    n = x.shape[0]
    r = _ROWS_PER_BLOCK
    t32 = target.astype(jnp.int32)
    out = pl.pallas_call(
        _tc_body,
        grid=(n // r,),
        in_specs=[
            pl.BlockSpec((n,), lambda i: (0,), memory_space=pltpu.SMEM),
            pl.BlockSpec((r, _SIZE), lambda i: (i, 0)),
        ],
        out_specs=pl.BlockSpec(memory_space=pltpu.SMEM),
        out_shape=jax.ShapeDtypeStruct((1, 1), jnp.float32),
    )(t32, x)
    return out[0, 0]


# speedup vs baseline: 2.4313x; 1.0282x over previous
import math

import jax
import jax.numpy as jnp
import numpy as np
from jax.experimental import pallas as pl
from jax.experimental.pallas import tpu as pltpu

_SIZE = 100000
_CONF = 0.9
_EPS = float(np.float32(0.1 / (_SIZE - 2)))
_H = _CONF * math.log(_CONF) + (_SIZE - 2) * _EPS * math.log(_EPS)
_ROWS_PER_BLOCK = 32


def _tc_body(t_ref, x_ref, o_ref):
    pid = pl.program_id(0)
    x = x_ref[...]
    rowsum = jnp.sum(x, axis=1)
    lane_iota = jax.lax.broadcasted_iota(jnp.int32, (1, 128), 1)
    contrib = 0.0
    for k in range(_ROWS_PER_BLOCK):
        tk = t_ref[pid * _ROWS_PER_BLOCK + k]
        wk = (tk != 0).astype(jnp.float32)
        col0 = pl.multiple_of((tk // 128) * 128, 128)
        window = x_ref[pl.ds(k, 1), pl.ds(col0, 128)]
        lane = tk - col0
        vk = jnp.sum(jnp.where(lane_iota == lane, window, 0.0))
        x0k = x_ref[k, 0]
        contrib += wk * (
            _H - (_CONF - _EPS) * vk - _EPS * (rowsum[k] - x0k)
        )

    @pl.when(pid == 0)
    def _init():
        o_ref[0, 0] = 0.0

    o_ref[0, 0] += contrib


def kernel(x, target):
    n = x.shape[0]
    r = _ROWS_PER_BLOCK
    t32 = target.astype(jnp.int32)
    out = pl.pallas_call(
        _tc_body,
        grid=(n // r,),
        in_specs=[
            pl.BlockSpec((n,), lambda i: (0,), memory_space=pltpu.SMEM),
            pl.BlockSpec((r, _SIZE), lambda i: (i, 0)),
        ],
        out_specs=pl.BlockSpec(memory_space=pltpu.SMEM),
        out_shape=jax.ShapeDtypeStruct((1, 1), jnp.float32),
    )(t32, x)
    return out[0, 0]
